# baseline (speedup 1.0000x reference)
import functools

import jax
import jax.numpy as jnp
from jax import lax
from jax.experimental import pallas as pl
from jax.experimental.pallas import tpu as pltpu
from jax.experimental.pallas import tpu_sc as plsc

_NC = 2
_NS = 16
_NW = _NC * _NS
_L = 16


def _zero_fill(ref, rows, width):
    zero = jnp.zeros((_L,), jnp.float32)

    def body(i, carry):
        for j in range(width // _L):
            ref[i, pl.ds(j * _L, _L)] = zero
        return carry

    lax.fori_loop(0, rows, body, 0)


@functools.lru_cache(maxsize=None)
def _make_agg(N, D, E, NP):
    C = 128
    n_chunks = E // C
    n_full = n_chunks // _NW
    n_extra = n_chunks % _NW
    assert n_full % 2 == 0
    per_tile = NP // _NS

    mesh = plsc.VectorSubcoreMesh(core_axis_name="c", subcore_axis_name="s")

    @functools.partial(
        pl.kernel,
        mesh=mesh,
        out_type=jax.ShapeDtypeStruct((_NC, NP, D), jnp.float32),
        scratch_types=[
            pltpu.VMEM((C,), jnp.int32),
            pltpu.VMEM((C,), jnp.int32),
            pltpu.VMEM((C,), jnp.int32),
            pltpu.VMEM((C,), jnp.int32),
            pltpu.VMEM((C, D), jnp.float32),
            pltpu.VMEM((C, D), jnp.float32),
            pltpu.VMEM_SHARED((NP, D), jnp.float32),
            pltpu.SemaphoreType.DMA,
            pltpu.SemaphoreType.DMA,
            pltpu.SemaphoreType.DMA,
            pltpu.SemaphoreType.DMA,
        ],
    )
    def agg(y_hbm, src_hbm, dst_hbm, out_hbm,
            src0, src1, dst0, dst1, rows0, rows1, acc_sh,
            gs0, gs1, ss0, ss1):
        cid = lax.axis_index("c")
        sid = lax.axis_index("s")
        wid = sid * _NC + cid

        srcv = (src0, src1)
        dstv = (dst0, dst1)
        rows = (rows0, rows1)
        gss = (gs0, gs1)
        sss = (ss0, ss1)

        def load_idx(q, bb):
            base = (wid + q * _NW) * C
            pltpu.sync_copy(src_hbm.at[pl.ds(base, C)], srcv[bb])
            pltpu.sync_copy(dst_hbm.at[pl.ds(base, C)], dstv[bb])

        def fire_gather(bb):
            pltpu.async_copy(y_hbm.at[srcv[bb]], rows[bb], gss[bb])

        def wait_gather(bb):
            pltpu.make_async_copy(y_hbm.at[srcv[bb]], rows[bb], gss[bb]).wait()

        def scatter(bb):
            pltpu.sync_copy(rows[bb], acc_sh.at[dstv[bb]], add=True)

        def fire_scatter(bb):
            pltpu.async_copy(rows[bb], acc_sh.at[dstv[bb]], sss[bb], add=True)

        def wait_scatter(bb):
            pltpu.make_async_copy(rows[bb], acc_sh.at[dstv[bb]], sss[bb]).wait()

        load_idx(0, 0)
        fire_gather(0)

        _zero_fill(rows1, C, D)
        row0 = sid * per_tile
        for k in range(per_tile // C):
            pltpu.sync_copy(rows1, acc_sh.at[pl.ds(row0 + k * C, C)])
        plsc.subcore_barrier()

        def pair(p, carry):
            for half in (0, 1):
                q = 2 * p + half
                bb = half
                b1 = 1 - half

                @pl.when(q + 1 < n_full)
                def _(q=q, b1=b1):
                    @pl.when(q >= 1)
                    def _():
                        wait_scatter(b1)

                    load_idx(q + 1, b1)
                    fire_gather(b1)

                wait_gather(bb)
                fire_scatter(bb)
            return carry

        lax.fori_loop(0, n_full // 2, pair, 0)
        wait_scatter(0)
        wait_scatter(1)

        @pl.when(wid < n_extra)
        def _():
            load_idx(n_full, 0)
            fire_gather(0)
            wait_gather(0)
            scatter(0)

        plsc.subcore_barrier()
        pltpu.sync_copy(acc_sh.at[pl.ds(row0, per_tile)],
                        out_hbm.at[cid, pl.ds(row0, per_tile)])

    return agg


@functools.lru_cache(maxsize=None)
def _make_deg(N, E, NP, D):
    C = 128
    n_chunks = E // C
    n_full = n_chunks // _NW
    n_extra = n_chunks % _NW
    per_tile = NP // _NS

    mesh = plsc.VectorSubcoreMesh(core_axis_name="c", subcore_axis_name="s")

    @functools.partial(
        pl.kernel,
        mesh=mesh,
        out_type=(jax.ShapeDtypeStruct((_NC, NP, D), jnp.float32),
                  jax.ShapeDtypeStruct((_NC, NP, D), jnp.float32)),
        scratch_types=[
            pltpu.VMEM((C,), jnp.int32),
            pltpu.VMEM((C, D), jnp.float32),
            pltpu.VMEM((C, D), jnp.float32),
            pltpu.VMEM_SHARED((NP, D), jnp.float32),
        ],
    )
    def deg(src_hbm, dst_hbm, dout_hbm, din_hbm,
            idx_v, ones_v, zeros_v, acc_sh):
        cid = lax.axis_index("c")
        sid = lax.axis_index("s")
        wid = sid * _NC + cid

        one = jnp.ones((_L,), jnp.float32)

        def fill(i, carry):
            for j in range(D // _L):
                ones_v[i, pl.ds(j * _L, _L)] = one
            return carry

        lax.fori_loop(0, C, fill, 0)
        _zero_fill(zeros_v, C, D)
        row0 = sid * per_tile

        def zero_acc():
            for k in range(per_tile // C):
                pltpu.sync_copy(zeros_v, acc_sh.at[pl.ds(row0 + k * C, C)])

        def scatter_pass(ind_hbm):
            def chunk(it):
                base = (wid + it * _NW) * C
                pltpu.sync_copy(ind_hbm.at[pl.ds(base, C)], idx_v)
                pltpu.sync_copy(ones_v, acc_sh.at[idx_v], add=True)

            def body(it, carry):
                chunk(it)
                return carry

            lax.fori_loop(0, n_full, body, 0)

            @pl.when(wid < n_extra)
            def _():
                chunk(n_full)

        zero_acc()
        plsc.subcore_barrier()
        scatter_pass(src_hbm)
        plsc.subcore_barrier()
        pltpu.sync_copy(acc_sh.at[pl.ds(row0, per_tile)],
                        dout_hbm.at[cid, pl.ds(row0, per_tile)])
        zero_acc()
        plsc.subcore_barrier()
        scatter_pass(dst_hbm)
        plsc.subcore_barrier()
        pltpu.sync_copy(acc_sh.at[pl.ds(row0, per_tile)],
                        din_hbm.at[cid, pl.ds(row0, per_tile)])

    return deg


def _norm(c0, c1):
    deg = c0[0, :, 0:1] + c1[0, :, 0:1]
    return lax.rsqrt(jnp.maximum(deg, 1.0))


def _pre_body(x_ref, d0_ref, d1_ref, w_ref, o_ref):
    ns = _norm(d0_ref[...], d1_ref[...])
    o_ref[...] = jnp.dot(x_ref[...] * ns, w_ref[...],
                         preferred_element_type=jnp.float32)


def _mid_body(a0_ref, a1_ref, di0_ref, di1_ref, do0_ref, do1_ref,
              b_ref, w_ref, o_ref):
    nd = _norm(di0_ref[...], di1_ref[...])
    ns = _norm(do0_ref[...], do1_ref[...])
    a = a0_ref[0] + a1_ref[0]
    t = jnp.maximum(a * nd + b_ref[...], 0.0)
    o_ref[...] = jnp.dot(t * ns, w_ref[...],
                         preferred_element_type=jnp.float32)


def _fin_body(a0_ref, a1_ref, di0_ref, di1_ref, b_ref, o_ref):
    i = pl.program_id(0)
    nd = _norm(di0_ref[...], di1_ref[...])
    a = a0_ref[0] + a1_ref[0]
    t = jnp.maximum(a * nd + b_ref[...], 0.0)
    m = jnp.max(t, axis=0, keepdims=True)

    @pl.when(i == 0)
    def _():
        o_ref[...] = m

    @pl.when(i > 0)
    def _():
        o_ref[...] = jnp.maximum(o_ref[...], m)


def _blk2(B, W):
    return pl.BlockSpec((B, W), lambda i: (i, 0))


def _blk3(B, W, c):
    return pl.BlockSpec((1, B, W), lambda i, _c=c: (_c, i, 0))


def _full(shape):
    nd = len(shape)
    return pl.BlockSpec(shape, lambda i: (0,) * nd)


def _pre_tc(x, dout2, W, B):
    N, D = x.shape
    return pl.pallas_call(
        _pre_body,
        grid=(N // B,),
        in_specs=[_blk2(B, D), _blk3(B, D, 0), _blk3(B, D, 1), _full((D, D))],
        out_specs=_blk2(B, D),
        out_shape=jax.ShapeDtypeStruct((N, D), jnp.float32),
    )(x, dout2, dout2, W)


def _mid_tc(N, a2, din2, dout2, b, W, B):
    D = a2.shape[-1]
    return pl.pallas_call(
        _mid_body,
        grid=(N // B,),
        in_specs=[_blk3(B, D, 0), _blk3(B, D, 1),
                  _blk3(B, D, 0), _blk3(B, D, 1),
                  _blk3(B, D, 0), _blk3(B, D, 1),
                  _full((1, D)), _full((D, D))],
        out_specs=_blk2(B, D),
        out_shape=jax.ShapeDtypeStruct((N, D), jnp.float32),
    )(a2, a2, din2, din2, dout2, dout2, b, W)


def _fin_tc(N, a2, din2, b, B):
    D = a2.shape[-1]
    return pl.pallas_call(
        _fin_body,
        grid=(N // B,),
        in_specs=[_blk3(B, D, 0), _blk3(B, D, 1),
                  _blk3(B, D, 0), _blk3(B, D, 1),
                  _full((1, D))],
        out_specs=_full((1, D)),
        out_shape=jax.ShapeDtypeStruct((1, D), jnp.float32),
    )(a2, a2, din2, din2, b)


def kernel(x1, x2, x3, g1, g2, g3, W1, b1, W2, b2, W3, b3):
    N, D = x1.shape
    E = g1.shape[1]
    B = 2000
    NP = 10240

    agg = _make_agg(N, D, E, NP)
    deg = _make_deg(N, E, NP, D)
    b1r = b1.reshape(1, D)
    b2r = b2.reshape(1, D)
    b3r = b3.reshape(1, D)

    maxes = []
    for x, g in ((x1, g1), (x2, g2), (x3, g3)):
        src = g[0]
        dst = g[1]
        dout2, din2 = deg(src, dst)
        y = _pre_tc(x, dout2, W1, B)
        a2 = agg(y, src, dst)
        y = _mid_tc(N, a2, din2, dout2, b1r, W2, B)
        a2 = agg(y, src, dst)
        y = _mid_tc(N, a2, din2, dout2, b2r, W3, B)
        a2 = agg(y, src, dst)
        maxes.append(_fin_tc(N, a2, din2, b3r, B))

    m = jnp.maximum(jnp.maximum(maxes[0], maxes[1]), maxes[2])
    return jnp.mean(m)


# speedup vs baseline: 1.1110x; 1.1110x over previous
import functools

import jax
import jax.numpy as jnp
from jax import lax
from jax.experimental import pallas as pl
from jax.experimental.pallas import tpu as pltpu
from jax.experimental.pallas import tpu_sc as plsc

_NC = 2
_NS = 16
_NW = _NC * _NS
_L = 16


def _zero_fill(ref, rows, width):
    zero = jnp.zeros((_L,), jnp.float32)

    def body(i, carry):
        for j in range(width // _L):
            ref[i, pl.ds(j * _L, _L)] = zero
        return carry

    lax.fori_loop(0, rows, body, 0)


@functools.lru_cache(maxsize=None)
def _make_agg(N, D, E, NP):
    C = 128
    n_chunks = E // C
    n_full = n_chunks // _NW
    n_extra = n_chunks % _NW
    assert n_full % 2 == 0
    per_tile = NP // _NS

    mesh = plsc.VectorSubcoreMesh(core_axis_name="c", subcore_axis_name="s")

    @functools.partial(
        pl.kernel,
        mesh=mesh,
        out_type=jax.ShapeDtypeStruct((_NC, NP, D), jnp.float32),
        scratch_types=[
            pltpu.VMEM((C,), jnp.int32),
            pltpu.VMEM((C,), jnp.int32),
            pltpu.VMEM((C,), jnp.int32),
            pltpu.VMEM((C,), jnp.int32),
            pltpu.VMEM((C, D), jnp.float32),
            pltpu.VMEM((C, D), jnp.float32),
            pltpu.VMEM_SHARED((NP, D), jnp.float32),
            pltpu.SemaphoreType.DMA,
            pltpu.SemaphoreType.DMA,
        ],
    )
    def agg(y_hbm, src_hbm, dst_hbm, out_hbm,
            src0, src1, dst0, dst1, rows0, rows1, acc_sh, gs0, gs1):
        cid = lax.axis_index("c")
        sid = lax.axis_index("s")
        wid = sid * _NC + cid

        srcv = (src0, src1)
        dstv = (dst0, dst1)
        rows = (rows0, rows1)
        gss = (gs0, gs1)

        def load_idx(q, bb):
            base = (wid + q * _NW) * C
            pltpu.sync_copy(src_hbm.at[pl.ds(base, C)], srcv[bb])
            pltpu.sync_copy(dst_hbm.at[pl.ds(base, C)], dstv[bb])

        def fire_gather(bb):
            pltpu.async_copy(y_hbm.at[srcv[bb]], rows[bb], gss[bb])

        def wait_gather(bb):
            pltpu.make_async_copy(y_hbm.at[srcv[bb]], rows[bb], gss[bb]).wait()

        def scatter(bb):
            pltpu.sync_copy(rows[bb], acc_sh.at[dstv[bb]], add=True)

        load_idx(0, 0)
        fire_gather(0)

        _zero_fill(rows1, C, D)
        row0 = sid * per_tile
        for k in range(per_tile // C):
            pltpu.sync_copy(rows1, acc_sh.at[pl.ds(row0 + k * C, C)])
        plsc.subcore_barrier()

        def pair(p, carry):
            for half in (0, 1):
                q = 2 * p + half
                bb = half
                b1 = 1 - half

                @pl.when(q + 1 < n_full)
                def _(q=q, b1=b1):
                    load_idx(q + 1, b1)
                    fire_gather(b1)

                wait_gather(bb)
                scatter(bb)
            return carry

        lax.fori_loop(0, n_full // 2, pair, 0)

        @pl.when(wid < n_extra)
        def _():
            load_idx(n_full, 0)
            fire_gather(0)
            wait_gather(0)
            scatter(0)

        plsc.subcore_barrier()
        pltpu.sync_copy(acc_sh.at[pl.ds(row0, per_tile)],
                        out_hbm.at[cid, pl.ds(row0, per_tile)])

    return agg


@functools.lru_cache(maxsize=None)
def _make_deg(N, E, NP, D):
    C = 128
    n_chunks = E // C
    n_full = n_chunks // _NW
    n_extra = n_chunks % _NW
    assert n_full % 2 == 0
    per_tile = NP // _NS

    mesh = plsc.VectorSubcoreMesh(core_axis_name="c", subcore_axis_name="s")

    @functools.partial(
        pl.kernel,
        mesh=mesh,
        out_type=(jax.ShapeDtypeStruct((_NC, NP, D), jnp.float32),
                  jax.ShapeDtypeStruct((_NC, NP, D), jnp.float32)),
        scratch_types=[
            pltpu.VMEM((C,), jnp.int32),
            pltpu.VMEM((C,), jnp.int32),
            pltpu.VMEM((C, D), jnp.float32),
            pltpu.VMEM((C, D), jnp.float32),
            pltpu.VMEM_SHARED((NP, D), jnp.float32),
            pltpu.SemaphoreType.DMA,
            pltpu.SemaphoreType.DMA,
        ],
    )
    def deg(src_hbm, dst_hbm, dout_hbm, din_hbm,
            idx0, idx1, ones_v, zeros_v, acc_sh, ss0, ss1):
        cid = lax.axis_index("c")
        sid = lax.axis_index("s")
        wid = sid * _NC + cid

        idxs = (idx0, idx1)
        sss = (ss0, ss1)
        one = jnp.ones((_L,), jnp.float32)

        def fill(i, carry):
            for j in range(D // _L):
                ones_v[i, pl.ds(j * _L, _L)] = one
            return carry

        lax.fori_loop(0, C, fill, 0)
        _zero_fill(zeros_v, C, D)
        row0 = sid * per_tile

        def zero_acc(acc_sh):
            for k in range(per_tile // C):
                pltpu.sync_copy(zeros_v, acc_sh.at[pl.ds(row0 + k * C, C)])

        def scatter_pass(ind_hbm, acc_sh):
            def load_idx(q, bb):
                base = (wid + q * _NW) * C
                pltpu.sync_copy(ind_hbm.at[pl.ds(base, C)], idxs[bb])

            def fire_scatter(bb):
                pltpu.async_copy(ones_v, acc_sh.at[idxs[bb]], sss[bb], add=True)

            def wait_scatter(bb):
                pltpu.make_async_copy(ones_v, acc_sh.at[idxs[bb]], sss[bb]).wait()

            load_idx(0, 0)

            def pair(p, carry):
                for half in (0, 1):
                    q = 2 * p + half
                    bb = half
                    b1 = 1 - half
                    fire_scatter(bb)

                    @pl.when(q + 1 < n_full)
                    def _(q=q, b1=b1):
                        @pl.when(q >= 1)
                        def _():
                            wait_scatter(b1)

                        load_idx(q + 1, b1)
                return carry

            lax.fori_loop(0, n_full // 2, pair, 0)
            wait_scatter(0)
            wait_scatter(1)

            @pl.when(wid < n_extra)
            def _():
                load_idx(n_full, 0)
                pltpu.sync_copy(ones_v, acc_sh.at[idxs[0]], add=True)

        if True:
            zero_acc(acc_sh)
            plsc.subcore_barrier()
            scatter_pass(src_hbm, acc_sh)
            plsc.subcore_barrier()
            pltpu.sync_copy(acc_sh.at[pl.ds(row0, per_tile)],
                            dout_hbm.at[cid, pl.ds(row0, per_tile)])
            zero_acc(acc_sh)
            plsc.subcore_barrier()
            scatter_pass(dst_hbm, acc_sh)
            plsc.subcore_barrier()
            pltpu.sync_copy(acc_sh.at[pl.ds(row0, per_tile)],
                            din_hbm.at[cid, pl.ds(row0, per_tile)])

    return deg


def _norm(c0, c1):
    deg = c0[0, :, 0:1] + c1[0, :, 0:1]
    return lax.rsqrt(jnp.maximum(deg, 1.0))


def _pre_body(x_ref, d0_ref, d1_ref, w_ref, o_ref):
    ns = _norm(d0_ref[...], d1_ref[...])
    o_ref[...] = jnp.dot(x_ref[...] * ns, w_ref[...],
                         preferred_element_type=jnp.float32)


def _mid_body(a0_ref, a1_ref, di0_ref, di1_ref, do0_ref, do1_ref,
              b_ref, w_ref, o_ref):
    nd = _norm(di0_ref[...], di1_ref[...])
    ns = _norm(do0_ref[...], do1_ref[...])
    a = a0_ref[0] + a1_ref[0]
    t = jnp.maximum(a * nd + b_ref[...], 0.0)
    o_ref[...] = jnp.dot(t * ns, w_ref[...],
                         preferred_element_type=jnp.float32)


def _fin_body(a0_ref, a1_ref, di0_ref, di1_ref, b_ref, o_ref):
    i = pl.program_id(0)
    nd = _norm(di0_ref[...], di1_ref[...])
    a = a0_ref[0] + a1_ref[0]
    t = jnp.maximum(a * nd + b_ref[...], 0.0)
    m = jnp.max(t, axis=0, keepdims=True)

    @pl.when(i == 0)
    def _():
        o_ref[...] = m

    @pl.when(i > 0)
    def _():
        o_ref[...] = jnp.maximum(o_ref[...], m)


def _blk2(B, W):
    return pl.BlockSpec((B, W), lambda i: (i, 0))


def _blk3(B, W, c):
    return pl.BlockSpec((1, B, W), lambda i, _c=c: (_c, i, 0))


def _full(shape):
    nd = len(shape)
    return pl.BlockSpec(shape, lambda i: (0,) * nd)


def _pre_tc(x, dout2, W, B):
    N, D = x.shape
    return pl.pallas_call(
        _pre_body,
        grid=(N // B,),
        in_specs=[_blk2(B, D), _blk3(B, D, 0), _blk3(B, D, 1), _full((D, D))],
        out_specs=_blk2(B, D),
        out_shape=jax.ShapeDtypeStruct((N, D), jnp.float32),
    )(x, dout2, dout2, W)


def _mid_tc(N, a2, din2, dout2, b, W, B):
    D = a2.shape[-1]
    return pl.pallas_call(
        _mid_body,
        grid=(N // B,),
        in_specs=[_blk3(B, D, 0), _blk3(B, D, 1),
                  _blk3(B, D, 0), _blk3(B, D, 1),
                  _blk3(B, D, 0), _blk3(B, D, 1),
                  _full((1, D)), _full((D, D))],
        out_specs=_blk2(B, D),
        out_shape=jax.ShapeDtypeStruct((N, D), jnp.float32),
    )(a2, a2, din2, din2, dout2, dout2, b, W)


def _fin_tc(N, a2, din2, b, B):
    D = a2.shape[-1]
    return pl.pallas_call(
        _fin_body,
        grid=(N // B,),
        in_specs=[_blk3(B, D, 0), _blk3(B, D, 1),
                  _blk3(B, D, 0), _blk3(B, D, 1),
                  _full((1, D))],
        out_specs=_full((1, D)),
        out_shape=jax.ShapeDtypeStruct((1, D), jnp.float32),
    )(a2, a2, din2, din2, b)


def kernel(x1, x2, x3, g1, g2, g3, W1, b1, W2, b2, W3, b3):
    N, D = x1.shape
    E = g1.shape[1]
    B = 2000
    NP = 10240

    agg = _make_agg(N, D, E, NP)
    deg = _make_deg(N, E, NP, D)
    b1r = b1.reshape(1, D)
    b2r = b2.reshape(1, D)
    b3r = b3.reshape(1, D)

    maxes = []
    for x, g in ((x1, g1), (x2, g2), (x3, g3)):
        src = g[0]
        dst = g[1]
        dout2, din2 = deg(src, dst)
        y = _pre_tc(x, dout2, W1, B)
        a2 = agg(y, src, dst)
        y = _mid_tc(N, a2, din2, dout2, b1r, W2, B)
        a2 = agg(y, src, dst)
        y = _mid_tc(N, a2, din2, dout2, b2r, W3, B)
        a2 = agg(y, src, dst)
        maxes.append(_fin_tc(N, a2, din2, b3r, B))

    m = jnp.maximum(jnp.maximum(maxes[0], maxes[1]), maxes[2])
    return jnp.mean(m)
